# Initial kernel scaffold; baseline (speedup 1.0000x reference)
import jax
import jax.numpy as jnp
from jax.experimental import pallas as pl


def kernel(x, edge_index, W1, b1, W2, b2):
    raise NotImplementedError("write your pallas kernel here")


# speedup vs baseline: 10.3642x; 10.3642x over previous
import functools

import jax
import jax.numpy as jnp
from jax import lax
from jax.experimental import pallas as pl
from jax.experimental.pallas import tpu as pltpu
from jax.experimental.pallas import tpu_sc as plsc

N_NODES = 10000
D = 128
N_EDGES = 320000
E_LOOP = N_EDGES + N_NODES
NC, NS, L = 2, 16, 16
NW = NC * NS
K = 128
CH = 82
E_PAD = NW * CH * K
TRASH = N_NODES
ROWS = 10240
RPT = ROWS // NS

_mesh = plsc.VectorSubcoreMesh(core_axis_name="c", subcore_axis_name="s")


def _zero_fill(ref, nrows, ncols):
    z = jnp.zeros((L,), jnp.float32)
    per_row = ncols // L

    def body(i, _):
        ref[i // per_row, pl.ds((i % per_row) * L, L)] = z
        return _

    lax.fori_loop(0, nrows * per_row, body, 0)


@functools.partial(
    pl.kernel,
    mesh=_mesh,
    out_type=jax.ShapeDtypeStruct((NC, ROWS, L), jnp.float32),
    scratch_types=[
        pltpu.VMEM((CH, K), jnp.int32),
        pltpu.VMEM((K, L), jnp.float32),
        pltpu.VMEM((L, L), jnp.float32),
        pltpu.VMEM_SHARED((ROWS, L), jnp.float32),
    ],
)
def _deg_kernel(dstw, out, dst_v, ones_v, zbuf, acc):
    c = lax.axis_index("c")
    s = lax.axis_index("s")
    wid = c * NS + s
    pltpu.sync_copy(dstw.at[wid], dst_v)

    one = jnp.full((L,), 1.0, jnp.float32)

    def fill_ones(i, _):
        ones_v[i, :] = one
        return _

    lax.fori_loop(0, K, fill_ones, 0)
    _zero_fill(zbuf, L, L)

    def zacc(i, _):
        pltpu.sync_copy(zbuf, acc.at[pl.ds(s * RPT + i * L, L)])
        return _

    lax.fori_loop(0, RPT // L, zacc, 0)
    plsc.subcore_barrier()

    def body(j, _):
        pltpu.sync_copy(ones_v, acc.at[dst_v.at[j]], add=True)
        return _

    lax.fori_loop(0, CH, body, 0)
    plsc.subcore_barrier()
    pltpu.sync_copy(acc.at[pl.ds(s * RPT, RPT)], out.at[c, pl.ds(s * RPT, RPT)])


@functools.partial(
    pl.kernel,
    mesh=_mesh,
    out_type=jax.ShapeDtypeStruct((NC, ROWS, D), jnp.float32),
    scratch_types=[
        pltpu.VMEM((CH, K), jnp.int32),
        pltpu.VMEM((CH, K), jnp.int32),
        pltpu.VMEM((K, D), jnp.float32),
        pltpu.VMEM((L, D), jnp.float32),
        pltpu.VMEM_SHARED((ROWS, D), jnp.float32),
    ],
)
def _agg_kernel(g, srcw, dstw, out, src_v, dst_v, rows_v, zbuf, acc):
    c = lax.axis_index("c")
    s = lax.axis_index("s")
    wid = c * NS + s
    pltpu.sync_copy(srcw.at[wid], src_v)
    pltpu.sync_copy(dstw.at[wid], dst_v)
    _zero_fill(zbuf, L, D)

    def zacc(i, _):
        pltpu.sync_copy(zbuf, acc.at[pl.ds(s * RPT + i * L, L)])
        return _

    lax.fori_loop(0, RPT // L, zacc, 0)
    plsc.subcore_barrier()

    def body(j, _):
        pltpu.sync_copy(g.at[src_v.at[j]], rows_v)
        pltpu.sync_copy(rows_v, acc.at[dst_v.at[j]], add=True)
        return _

    lax.fori_loop(0, CH, body, 0)
    plsc.subcore_barrier()
    pltpu.sync_copy(acc.at[pl.ds(s * RPT, RPT)], out.at[c, pl.ds(s * RPT, RPT)])


def _tc_call(body, n_out, *args):
    return pl.pallas_call(
        body,
        out_shape=jax.ShapeDtypeStruct((N_NODES, D), jnp.float32),
    )(*args)


def _dinv(degw_ref):
    deg = degw_ref[0, :N_NODES, 0:1] + degw_ref[1, :N_NODES, 0:1]
    return lax.rsqrt(deg)


def _tc1_body(x_ref, w1_ref, degw_ref, g_ref):
    dinv = _dinv(degw_ref)
    h = lax.dot_general(x_ref[...], w1_ref[...], (((1,), (1,)), ((), ())),
                        preferred_element_type=jnp.float32)
    g_ref[...] = h * dinv


def _tc2_body(acc_ref, degw_ref, b1_ref, w2_ref, g_ref):
    dinv = _dinv(degw_ref)
    ssum = acc_ref[0, :N_NODES, :] + acc_ref[1, :N_NODES, :]
    a1 = jnp.maximum(ssum * dinv + b1_ref[...], 0.0)
    h2 = lax.dot_general(a1, w2_ref[...], (((1,), (1,)), ((), ())),
                         preferred_element_type=jnp.float32)
    g_ref[...] = h2 * dinv


def _tc3_body(acc_ref, degw_ref, b2_ref, out_ref):
    dinv = _dinv(degw_ref)
    ssum = acc_ref[0, :N_NODES, :] + acc_ref[1, :N_NODES, :]
    out_ref[...] = ssum * dinv + b2_ref[...]


def kernel(x, edge_index, W1, b1, W2, b2):
    ei = edge_index.astype(jnp.int32)
    loop = jnp.arange(N_NODES, dtype=jnp.int32)
    pad = E_PAD - E_LOOP
    src = jnp.concatenate([ei[0], loop, jnp.zeros((pad,), jnp.int32)])
    dst = jnp.concatenate([ei[1], loop, jnp.full((pad,), TRASH, jnp.int32)])
    srcw = src.reshape(NW, CH, K)
    dstw = dst.reshape(NW, CH, K)
    b1r = b1.reshape(1, D)
    b2r = b2.reshape(1, D)

    degw = _deg_kernel(dstw)
    g1 = _tc_call(_tc1_body, 1, x, W1, degw)
    acc1 = _agg_kernel(g1, srcw, dstw)
    g2 = _tc_call(_tc2_body, 1, acc1, degw, b1r, W2)
    acc2 = _agg_kernel(g2, srcw, dstw)
    out = _tc_call(_tc3_body, 1, acc2, degw, b2r)
    return out


# speedup vs baseline: 11.7185x; 1.1307x over previous
import functools

import jax
import jax.numpy as jnp
from jax import lax
from jax.experimental import pallas as pl
from jax.experimental.pallas import tpu as pltpu
from jax.experimental.pallas import tpu_sc as plsc

N_NODES = 10000
D = 128
N_EDGES = 320000
E_LOOP = N_EDGES + N_NODES
NC, NS, L = 2, 16, 16
NW = NC * NS
K = 128
CH = 82
E_PAD = NW * CH * K
TRASH = N_NODES
ROWS = 10240
RPT = ROWS // NS

_mesh = plsc.VectorSubcoreMesh(core_axis_name="c", subcore_axis_name="s")


def _zero_fill(ref, nrows, ncols):
    z = jnp.zeros((L,), jnp.float32)
    per_row = ncols // L

    def body(i, _):
        ref[i // per_row, pl.ds((i % per_row) * L, L)] = z
        return _

    lax.fori_loop(0, nrows * per_row, body, 0)


@functools.partial(
    pl.kernel,
    mesh=_mesh,
    out_type=jax.ShapeDtypeStruct((NC, ROWS, L), jnp.float32),
    scratch_types=[
        pltpu.VMEM((CH, K), jnp.int32),
        pltpu.VMEM((K, L), jnp.float32),
        pltpu.VMEM((L, L), jnp.float32),
        pltpu.VMEM_SHARED((ROWS, L), jnp.float32),
    ],
)
def _deg_kernel(dstw, out, dst_v, ones_v, zbuf, acc):
    c = lax.axis_index("c")
    s = lax.axis_index("s")
    wid = c * NS + s
    pltpu.sync_copy(dstw.at[wid], dst_v)

    one = jnp.full((L,), 1.0, jnp.float32)

    def fill_ones(i, _):
        ones_v[i, :] = one
        return _

    lax.fori_loop(0, K, fill_ones, 0)
    _zero_fill(zbuf, L, L)

    def zacc(i, _):
        pltpu.sync_copy(zbuf, acc.at[pl.ds(s * RPT + i * L, L)])
        return _

    lax.fori_loop(0, RPT // L, zacc, 0)
    plsc.subcore_barrier()

    def body(j, _):
        pltpu.sync_copy(ones_v, acc.at[dst_v.at[j]], add=True)
        return _

    lax.fori_loop(0, CH, body, 0)
    plsc.subcore_barrier()
    pltpu.sync_copy(acc.at[pl.ds(s * RPT, RPT)], out.at[c, pl.ds(s * RPT, RPT)])


@functools.partial(
    pl.kernel,
    mesh=_mesh,
    out_type=jax.ShapeDtypeStruct((NC, ROWS, D), jnp.float32),
    scratch_types=[
        pltpu.VMEM((CH, K), jnp.int32),
        pltpu.VMEM((2, K), jnp.int32),
        pltpu.VMEM((2, K, D), jnp.float32),
        pltpu.VMEM_SHARED((ROWS, D), jnp.float32),
        pltpu.SemaphoreType.DMA,
        pltpu.SemaphoreType.DMA,
    ],
)
def _agg_kernel(g, srcw, dstw, out, src_v, dst_v, rows_v, acc, gsem, isem):
    c = lax.axis_index("c")
    s = lax.axis_index("s")
    wid = c * NS + s
    pltpu.sync_copy(srcw.at[wid], src_v)
    zbuf = rows_v.at[0].at[pl.ds(0, L)]
    _zero_fill(zbuf, L, D)

    def zacc(i, _):
        pltpu.sync_copy(zbuf, acc.at[pl.ds(s * RPT + i * L, L)])
        return _

    lax.fori_loop(0, RPT // L, zacc, 0)
    plsc.subcore_barrier()

    def issue(j, b):
        pltpu.async_copy(g.at[src_v.at[j]], rows_v.at[b], gsem)
        pltpu.async_copy(dstw.at[wid, j], dst_v.at[b], isem)

    issue(0, 0)
    issue(1, 1)

    def body(j, _):
        b = lax.rem(j, 2)
        pltpu.make_async_copy(g.at[src_v.at[j]], rows_v.at[b], gsem).wait()
        pltpu.make_async_copy(dstw.at[wid, j], dst_v.at[b], isem).wait()
        pltpu.sync_copy(rows_v.at[b], acc.at[dst_v.at[b]], add=True)

        @pl.when(j + 2 < CH)
        def _next():
            issue(j + 2, b)

        return _

    lax.fori_loop(0, CH, body, 0)
    plsc.subcore_barrier()
    pltpu.sync_copy(acc.at[pl.ds(s * RPT, RPT)], out.at[c, pl.ds(s * RPT, RPT)])


def _tc_call(body, n_out, *args):
    return pl.pallas_call(
        body,
        out_shape=jax.ShapeDtypeStruct((N_NODES, D), jnp.float32),
    )(*args)


def _dinv(degw_ref):
    deg = degw_ref[0, :N_NODES, 0:1] + degw_ref[1, :N_NODES, 0:1]
    return lax.rsqrt(deg)


def _tc1_body(x_ref, w1_ref, degw_ref, g_ref):
    dinv = _dinv(degw_ref)
    h = lax.dot_general(x_ref[...], w1_ref[...], (((1,), (1,)), ((), ())),
                        preferred_element_type=jnp.float32)
    g_ref[...] = h * dinv


def _tc2_body(acc_ref, degw_ref, b1_ref, w2_ref, g_ref):
    dinv = _dinv(degw_ref)
    ssum = acc_ref[0, :N_NODES, :] + acc_ref[1, :N_NODES, :]
    a1 = jnp.maximum(ssum * dinv + b1_ref[...], 0.0)
    h2 = lax.dot_general(a1, w2_ref[...], (((1,), (1,)), ((), ())),
                         preferred_element_type=jnp.float32)
    g_ref[...] = h2 * dinv


def _tc3_body(acc_ref, degw_ref, b2_ref, out_ref):
    dinv = _dinv(degw_ref)
    ssum = acc_ref[0, :N_NODES, :] + acc_ref[1, :N_NODES, :]
    out_ref[...] = ssum * dinv + b2_ref[...]


def kernel(x, edge_index, W1, b1, W2, b2):
    ei = edge_index.astype(jnp.int32)
    loop = jnp.arange(N_NODES, dtype=jnp.int32)
    pad = E_PAD - E_LOOP
    src = jnp.concatenate([ei[0], loop, jnp.zeros((pad,), jnp.int32)])
    dst = jnp.concatenate([ei[1], loop, jnp.full((pad,), TRASH, jnp.int32)])
    srcw = src.reshape(NW, CH, K)
    dstw = dst.reshape(NW, CH, K)
    b1r = b1.reshape(1, D)
    b2r = b2.reshape(1, D)

    degw = _deg_kernel(dstw)
    g1 = _tc_call(_tc1_body, 1, x, W1, degw)
    acc1 = _agg_kernel(g1, srcw, dstw)
    g2 = _tc_call(_tc2_body, 1, acc1, degw, b1r, W2)
    acc2 = _agg_kernel(g2, srcw, dstw)
    out = _tc_call(_tc3_body, 1, acc2, degw, b2r)
    return out


# speedup vs baseline: 12.2428x; 1.0447x over previous
import functools

import jax
import jax.numpy as jnp
from jax import lax
from jax.experimental import pallas as pl
from jax.experimental.pallas import tpu as pltpu
from jax.experimental.pallas import tpu_sc as plsc

N_NODES = 10000
D = 128
N_EDGES = 320000
E_LOOP = N_EDGES + N_NODES
NC, NS, L = 2, 16, 16
NW = NC * NS
K = 128
CH = 82
E_PAD = NW * CH * K
TRASH = N_NODES
ROWS = 10240
RPT = ROWS // NS

_mesh = plsc.VectorSubcoreMesh(core_axis_name="c", subcore_axis_name="s")


def _zero_fill(ref, nrows, ncols):
    z = jnp.zeros((L,), jnp.float32)
    per_row = ncols // L

    def body(i, _):
        ref[i // per_row, pl.ds((i % per_row) * L, L)] = z
        return _

    lax.fori_loop(0, nrows * per_row, body, 0)


@functools.partial(
    pl.kernel,
    mesh=_mesh,
    out_type=jax.ShapeDtypeStruct((NC, ROWS, L), jnp.float32),
    scratch_types=[
        pltpu.VMEM((CH, K), jnp.int32),
        pltpu.VMEM((K, L), jnp.float32),
        pltpu.VMEM((L, L), jnp.float32),
        pltpu.VMEM_SHARED((ROWS, L), jnp.float32),
    ],
)
def _deg_kernel(dstw, out, dst_v, ones_v, zbuf, acc):
    c = lax.axis_index("c")
    s = lax.axis_index("s")
    wid = c * NS + s
    pltpu.sync_copy(dstw.at[wid], dst_v)

    one = jnp.full((L,), 1.0, jnp.float32)

    def fill_ones(i, _):
        ones_v[i, :] = one
        return _

    lax.fori_loop(0, K, fill_ones, 0)
    _zero_fill(zbuf, L, L)

    def zacc(i, _):
        pltpu.sync_copy(zbuf, acc.at[pl.ds(s * RPT + i * L, L)])
        return _

    lax.fori_loop(0, RPT // L, zacc, 0)
    plsc.subcore_barrier()

    def body(j, _):
        pltpu.sync_copy(ones_v, acc.at[dst_v.at[j]], add=True)
        return _

    lax.fori_loop(0, CH, body, 0)
    plsc.subcore_barrier()
    pltpu.sync_copy(acc.at[pl.ds(s * RPT, RPT)], out.at[c, pl.ds(s * RPT, RPT)])


@functools.partial(
    pl.kernel,
    mesh=_mesh,
    out_type=jax.ShapeDtypeStruct((NC, ROWS, D), jnp.float32),
    scratch_types=[
        pltpu.VMEM((CH, K), jnp.int32),
        pltpu.VMEM((2, K), jnp.int32),
        pltpu.VMEM((2, K, D), jnp.float32),
        pltpu.VMEM_SHARED((ROWS, D), jnp.float32),
        pltpu.SemaphoreType.DMA,
        pltpu.SemaphoreType.DMA,
    ],
)
def _agg_kernel(g, srcw, dstw, out, src_v, dst_v, rows_v, acc, gsem, isem):
    c = lax.axis_index("c")
    s = lax.axis_index("s")
    wid = c * NS + s
    pltpu.sync_copy(srcw.at[wid], src_v)
    zbuf = rows_v.at[0].at[pl.ds(0, L)]
    _zero_fill(zbuf, L, D)

    def zacc(i, _):
        pltpu.sync_copy(zbuf, acc.at[pl.ds(s * RPT + i * L, L)])
        return _

    lax.fori_loop(0, RPT // L, zacc, 0)
    plsc.subcore_barrier()

    NSUB = 4
    SUB = K // NSUB

    def issue(j, b):
        for i in range(NSUB):
            pltpu.async_copy(
                g.at[src_v.at[j].at[pl.ds(i * SUB, SUB)]],
                rows_v.at[b].at[pl.ds(i * SUB, SUB)], gsem)
        pltpu.async_copy(dstw.at[wid, j], dst_v.at[b], isem)

    issue(0, 0)
    issue(1, 1)

    def body(j, _):
        b = lax.rem(j, 2)
        for i in range(NSUB):
            pltpu.make_async_copy(
                g.at[src_v.at[j].at[pl.ds(i * SUB, SUB)]],
                rows_v.at[b].at[pl.ds(i * SUB, SUB)], gsem).wait()
        pltpu.make_async_copy(dstw.at[wid, j], dst_v.at[b], isem).wait()
        pltpu.sync_copy(rows_v.at[b], acc.at[dst_v.at[b]], add=True)

        @pl.when(j + 2 < CH)
        def _next():
            issue(j + 2, b)

        return _

    lax.fori_loop(0, CH, body, 0)
    plsc.subcore_barrier()
    pltpu.sync_copy(acc.at[pl.ds(s * RPT, RPT)], out.at[c, pl.ds(s * RPT, RPT)])


def _tc_call(body, n_out, *args):
    return pl.pallas_call(
        body,
        out_shape=jax.ShapeDtypeStruct((N_NODES, D), jnp.float32),
    )(*args)


def _dinv(degw_ref):
    deg = degw_ref[0, :N_NODES, 0:1] + degw_ref[1, :N_NODES, 0:1]
    return lax.rsqrt(deg)


def _tc1_body(x_ref, w1_ref, degw_ref, g_ref):
    dinv = _dinv(degw_ref)
    h = lax.dot_general(x_ref[...], w1_ref[...], (((1,), (1,)), ((), ())),
                        preferred_element_type=jnp.float32)
    g_ref[...] = h * dinv


def _tc2_body(acc_ref, degw_ref, b1_ref, w2_ref, g_ref):
    dinv = _dinv(degw_ref)
    ssum = acc_ref[0, :N_NODES, :] + acc_ref[1, :N_NODES, :]
    a1 = jnp.maximum(ssum * dinv + b1_ref[...], 0.0)
    h2 = lax.dot_general(a1, w2_ref[...], (((1,), (1,)), ((), ())),
                         preferred_element_type=jnp.float32)
    g_ref[...] = h2 * dinv


def _tc3_body(acc_ref, degw_ref, b2_ref, out_ref):
    dinv = _dinv(degw_ref)
    ssum = acc_ref[0, :N_NODES, :] + acc_ref[1, :N_NODES, :]
    out_ref[...] = ssum * dinv + b2_ref[...]


def kernel(x, edge_index, W1, b1, W2, b2):
    ei = edge_index.astype(jnp.int32)
    loop = jnp.arange(N_NODES, dtype=jnp.int32)
    pad = E_PAD - E_LOOP
    src = jnp.concatenate([ei[0], loop, jnp.zeros((pad,), jnp.int32)])
    dst = jnp.concatenate([ei[1], loop, jnp.full((pad,), TRASH, jnp.int32)])
    srcw = src.reshape(NW, CH, K)
    dstw = dst.reshape(NW, CH, K)
    b1r = b1.reshape(1, D)
    b2r = b2.reshape(1, D)

    degw = _deg_kernel(dstw)
    g1 = _tc_call(_tc1_body, 1, x, W1, degw)
    acc1 = _agg_kernel(g1, srcw, dstw)
    g2 = _tc_call(_tc2_body, 1, acc1, degw, b1r, W2)
    acc2 = _agg_kernel(g2, srcw, dstw)
    out = _tc_call(_tc3_body, 1, acc2, degw, b2r)
    return out
